# Initial kernel scaffold; baseline (speedup 1.0000x reference)
import jax
import jax.numpy as jnp
from jax.experimental import pallas as pl


def kernel(h_node, pos_node, h_edge, edge_index, node_time, edge_time, params):
    raise NotImplementedError("write your pallas kernel here")


# speedup vs baseline: 1.0004x; 1.0004x over previous
import functools

import jax
import jax.numpy as jnp
from jax.experimental import pallas as pl

_N_TILE = 1000
_E_TILE = 2000
_NUM_GAUSS = 16
_CUTOFF = 10.0


def _rowstage(body_fn, tile, nrows, rows_list, weights_list, out_widths):
    grid = (nrows // tile,)
    n_rows_in = len(rows_list)
    n_w = len(weights_list)

    def kbody(*refs):
        in_refs = refs[: n_rows_in + n_w]
        out_refs = refs[n_rows_in + n_w:]
        rows = [r[...] for r in in_refs[:n_rows_in]]
        ws = [r[...] for r in in_refs[n_rows_in:]]
        outs = body_fn(rows, ws)
        for oref, o in zip(out_refs, outs):
            oref[...] = o

    in_specs = (
        [pl.BlockSpec((tile, a.shape[1]), lambda i: (i, 0)) for a in rows_list]
        + [pl.BlockSpec(w.shape, lambda i: (0, 0)) for w in weights_list]
    )
    out_specs = [pl.BlockSpec((tile, w), lambda i: (i, 0)) for w in out_widths]
    out_shape = [jax.ShapeDtypeStruct((nrows, w), jnp.float32) for w in out_widths]
    return pl.pallas_call(
        kbody,
        grid=grid,
        in_specs=in_specs,
        out_specs=out_specs,
        out_shape=out_shape,
    )(*rows_list, *weights_list)


def _dot(x, w):
    return jnp.dot(x, w, preferred_element_type=jnp.float32)


def _layer_norm(x, g, b):
    mu = jnp.mean(x, axis=-1, keepdims=True)
    var = jnp.mean((x - mu) * (x - mu), axis=-1, keepdims=True)
    return (x - mu) / jnp.sqrt(var + 1e-5) * g + b


def _b2(b):
    return b.reshape(1, -1)


def _node1_body(rows, ws):
    (x, n_t) = rows
    (nn_W1, nn_b1, nn_W2, nn_b2,
     g_W1x, g_W1t, g_b1,
     nlL_W, glx_W, gl_b1, nfl_W, nfl_b,
     nlR_W, grx_W, gr_b1, nfr_W, nfr_b,
     cen_W, cen_b) = ws
    h_nn = _dot(jax.nn.relu(_dot(x, nn_W1) + nn_b1), nn_W2) + nn_b2
    g1 = _dot(x, g_W1x) + _dot(n_t, g_W1t) + g_b1
    srcF = jnp.concatenate(
        [_dot(x, nlL_W), _dot(x, glx_W) + gl_b1, _dot(x, nfl_W) + nfl_b], axis=1)
    dstF = jnp.concatenate(
        [h_nn, g1, _dot(x, nlR_W), _dot(x, grx_W) + gr_b1,
         _dot(x, nfr_W) + nfr_b], axis=1)
    cent = _dot(x, cen_W) + cen_b
    return srcF, dstF, cent


def _edge1_body(rows, ws):
    (h_edge, e_t, p_s, p_d, srcF, dstF) = rows
    (gauss_off,
     We_e, We_g, be,
     en_W1, en_b1, en_W2, en_b2,
     mn_W, mn_b,
     g_W1e, g_W2, g_b2,
     bl_Wb, il_W1, il_b1, il_W2, il_b2, gl_W1b, gl_W1t, gl_W2, gl_b2,
     br_Wb, ir_W1, ir_b1, ir_W2, ir_b2, gr_W1b, gr_W1t, gr_W2, gr_b2,
     sf_W, sf_b) = ws
    rel = p_s - p_d
    dist = jnp.sqrt(jnp.sum(rel * rel, axis=1, keepdims=True))
    coeff = -0.5 / (_CUTOFF / (_NUM_GAUSS - 1)) ** 2
    diff = dist - gauss_off
    gauss = jnp.exp(coeff * diff * diff)
    h_e = _dot(h_edge, We_e) + _dot(gauss, We_g) + be

    h_en = _dot(jax.nn.relu(_dot(h_e, en_W1) + en_b1), en_W2) + en_b2
    hn_d = dstF[:, :128]
    g1_d = dstF[:, 128:256]
    nlR = dstF[:, 256:288]
    grN = dstF[:, 288:320]
    nffr = dstF[:, 320:336]
    nlL = srcF[:, :32]
    glN = srcF[:, 32:64]
    nffl = srcF[:, 64:80]
    msg = _dot(h_en * hn_d, mn_W) + mn_b
    gate = _dot(jax.nn.relu(_dot(h_e, g_W1e) + g1_d), g_W2) + g_b2
    msg = msg * jax.nn.sigmoid(gate)

    bf_l = _dot(h_e, bl_Wb)
    inter_l = _dot(jax.nn.relu(_dot(bf_l * nlL, il_W1) + il_b1), il_W2) + il_b2
    gate_l = _dot(jax.nn.relu(_dot(h_e, gl_W1b) + _dot(e_t, gl_W1t) + glN),
                  gl_W2) + gl_b2
    mlr = inter_l * jax.nn.sigmoid(gate_l)

    bf_r = _dot(h_e, br_Wb)
    inter_r = _dot(jax.nn.relu(_dot(bf_r * nlR, ir_W1) + ir_b1), ir_W2) + ir_b2
    gate_r = _dot(jax.nn.relu(_dot(h_e, gr_W1b) + _dot(e_t, gr_W1t) + grN),
                  gr_W2) + gr_b2
    mrr = inter_r * jax.nn.sigmoid(gate_r)

    partial = nffl + nffr + _dot(h_e, sf_W) + sf_b
    return msg, mlr, mrr, h_e, partial, rel, dist


def _node2_body(rows, ws):
    (x, cent, aggr) = rows
    (ln_g, ln_b, out_W, out_b,
     L_W1, L_b1, L_W2, L_b2,
     R_W1, R_b1, R_W2, R_b2) = ws
    o = _layer_norm(cent + aggr, ln_g, ln_b)
    h_new = x + _dot(jax.nn.relu(o), out_W) + out_b
    lf = _dot(jax.nn.relu(_dot(h_new, L_W1) + L_b1), L_W2) + L_b2
    rf = _dot(jax.nn.relu(_dot(h_new, R_W1) + R_b1), R_W2) + R_b2
    return h_new, lf, rf


def _edge2_body(rows, ws):
    (sl_s, sr_d, partial, h_e, lf_s, rf_d, e_t, rel, dist) = rows
    (ln_g, ln_b, out_W, out_b,
     pe_Wb, pe_Wn, pi_W1, pi_b1, pi_W2, pi_b2,
     pg_W1, pg_b1, pg_W2, pg_b2) = ws
    h = _layer_norm(sl_s + sr_d + partial, ln_g, ln_b)
    h_e_new = h_e + _dot(jax.nn.relu(h), out_W) + out_b

    lr = lf_s * rf_d
    bf = _dot(h_e_new, pe_Wb)
    nf = _dot(lr, pe_Wn)
    inter = _dot(jax.nn.relu(_dot(bf * nf, pi_W1) + pi_b1), pi_W2) + pi_b2
    gin = jnp.concatenate([h_e_new, lr, e_t], axis=1)
    gate = _dot(jax.nn.relu(_dot(gin, pg_W1) + pg_b1), pg_W2) + pg_b2
    w = inter * jax.nn.sigmoid(gate)
    force = w * rel / dist / (dist + 1.0)
    return h_e_new, force


def _gather(tab, idx):
    return jnp.take(tab, idx, axis=0)


def _segsum(vals, idx, n):
    return jax.ops.segment_sum(vals, idx, num_segments=n)


def _prep_block(params_i):
    nb = params_i['nb']
    eb = params_i['eb']
    pb = params_i['pb']
    ee = params_i['ee']
    ND = 128
    ED = 16

    gW1 = nb['gate']['l1']['W']
    w_node1 = [
        nb['node_net']['l1']['W'], _b2(nb['node_net']['l1']['b']),
        nb['node_net']['l2']['W'], _b2(nb['node_net']['l2']['b']),
        gW1[ED:ED + ND], gW1[ED + ND:], _b2(nb['gate']['l1']['b']),
        eb['ffn_left']['node_linear']['W'],
        eb['ffn_left']['gate']['l1']['W'][ED:ED + ND],
        _b2(eb['ffn_left']['gate']['l1']['b']),
        eb['node_ffn_left']['W'], _b2(eb['node_ffn_left']['b']),
        eb['ffn_right']['node_linear']['W'],
        eb['ffn_right']['gate']['l1']['W'][ED:ED + ND],
        _b2(eb['ffn_right']['gate']['l1']['b']),
        eb['node_ffn_right']['W'], _b2(eb['node_ffn_right']['b']),
        nb['centroid']['W'], _b2(nb['centroid']['b']),
    ]

    gauss_off = jnp.linspace(0.0, _CUTOFF, _NUM_GAUSS,
                             dtype=jnp.float32).reshape(1, -1)
    eeW = ee['W']
    w_edge1 = [
        gauss_off,
        eeW[:ED], eeW[ED:], _b2(ee['b']),
        nb['edge_net']['l1']['W'], _b2(nb['edge_net']['l1']['b']),
        nb['edge_net']['l2']['W'], _b2(nb['edge_net']['l2']['b']),
        nb['msg_net']['W'], _b2(nb['msg_net']['b']),
        gW1[:ED], nb['gate']['l2']['W'], _b2(nb['gate']['l2']['b']),
        eb['ffn_left']['bond_linear']['W'],
        eb['ffn_left']['inter']['l1']['W'], _b2(eb['ffn_left']['inter']['l1']['b']),
        eb['ffn_left']['inter']['l2']['W'], _b2(eb['ffn_left']['inter']['l2']['b']),
        eb['ffn_left']['gate']['l1']['W'][:ED],
        eb['ffn_left']['gate']['l1']['W'][ED + ND:],
        eb['ffn_left']['gate']['l2']['W'], _b2(eb['ffn_left']['gate']['l2']['b']),
        eb['ffn_right']['bond_linear']['W'],
        eb['ffn_right']['inter']['l1']['W'], _b2(eb['ffn_right']['inter']['l1']['b']),
        eb['ffn_right']['inter']['l2']['W'], _b2(eb['ffn_right']['inter']['l2']['b']),
        eb['ffn_right']['gate']['l1']['W'][:ED],
        eb['ffn_right']['gate']['l1']['W'][ED + ND:],
        eb['ffn_right']['gate']['l2']['W'], _b2(eb['ffn_right']['gate']['l2']['b']),
        eb['self_ffn']['W'], _b2(eb['self_ffn']['b']),
    ]

    w_node2 = [
        _b2(nb['ln']['g']), _b2(nb['ln']['b']),
        nb['out']['W'], _b2(nb['out']['b']),
        pb['left']['l1']['W'], _b2(pb['left']['l1']['b']),
        pb['left']['l2']['W'], _b2(pb['left']['l2']['b']),
        pb['right']['l1']['W'], _b2(pb['right']['l1']['b']),
        pb['right']['l2']['W'], _b2(pb['right']['l2']['b']),
    ]

    el = pb['edge_lin']
    w_edge2 = [
        _b2(eb['ln']['g']), _b2(eb['ln']['b']),
        eb['out']['W'], _b2(eb['out']['b']),
        el['bond_linear']['W'], el['node_linear']['W'],
        el['inter']['l1']['W'], _b2(el['inter']['l1']['b']),
        el['inter']['l2']['W'], _b2(el['inter']['l2']['b']),
        el['gate']['l1']['W'], _b2(el['gate']['l1']['b']),
        el['gate']['l2']['W'], _b2(el['gate']['l2']['b']),
    ]
    return w_node1, w_edge1, w_node2, w_edge2


def kernel(h_node, pos_node, h_edge, edge_index, node_time, edge_time, params):
    n = h_node.shape[0]
    e = h_edge.shape[0]
    src = edge_index[0]
    dst = edge_index[1]

    num_blocks = len(params['node_blocks'])
    for i in range(num_blocks):
        pk = {'nb': params['node_blocks'][i], 'eb': params['edge_blocks'][i],
              'pb': params['pos_blocks'][i], 'ee': params['edge_embs'][i]}
        w_node1, w_edge1, w_node2, w_edge2 = _prep_block(pk)

        srcF, dstF, cent = _rowstage(
            _node1_body, _N_TILE, n, [h_node, node_time], w_node1,
            [80, 336, 128])

        pos_s = _gather(pos_node, src)
        pos_d = _gather(pos_node, dst)
        srcF_e = _gather(srcF, src)
        dstF_e = _gather(dstF, dst)

        msg, mlr, mrr, h_e, partial, rel, dist = _rowstage(
            _edge1_body, _E_TILE, e,
            [h_edge, edge_time, pos_s, pos_d, srcF_e, dstF_e], w_edge1,
            [128, 16, 16, 16, 16, 3, 1])

        aggr = _segsum(msg, src, n)
        seg_l = _segsum(mlr, dst, n)
        seg_r = _segsum(mrr, src, n)

        h_node, lf, rf = _rowstage(
            _node2_body, _N_TILE, n, [h_node, cent, aggr], w_node2,
            [128, 16, 16])

        sl_s = _gather(seg_l, src)
        sr_d = _gather(seg_r, dst)
        lf_s = _gather(lf, src)
        rf_d = _gather(rf, dst)

        h_edge, force = _rowstage(
            _edge2_body, _E_TILE, e,
            [sl_s, sr_d, partial, h_e, lf_s, rf_d, edge_time, rel, dist],
            w_edge2, [16, 3])

        pos_node = pos_node + _segsum(force, src, n)

    return (h_node, pos_node, h_edge)
